# baseline (speedup 1.0000x reference)
import jax
import jax.numpy as jnp
from jax import lax
from jax.experimental import pallas as pl
from jax.experimental.pallas import tpu as pltpu

_B = 16
_C = 16
_SZ = ((64, 64), (32, 32), (16, 16), (8, 8), (4, 4))
_NPB = 5456
_N = _B * _NPB
_K = int(_N * 0.01)
_OFF = (0, 4096, 5120, 5376, 5440)


def _smooth_l1(x, y):
    d = jnp.abs(x - y)
    return jnp.where(d < 1.0, 0.5 * d * d, d - 0.5)


def _softplus_p(x):
    e = jnp.exp(-jnp.abs(x))
    a = 1.0 + e
    p = jnp.where(x >= 0.0, 1.0, e) / a
    sp = jnp.maximum(x, 0.0) + jnp.log(a)
    return sp, p


def _pack_rows(val, sent):
    del sent
    h = val.shape[1]
    w = val.shape[2]
    group = 128 // w
    pieces = [val[:, j * (h // group):(j + 1) * (h // group), :]
              for j in range(group)]
    return jnp.concatenate(pieces, axis=2)


def _packed_idx(shape, l):
    yi = lax.broadcasted_iota(jnp.int32, shape, 1)
    xi = lax.broadcasted_iota(jnp.int32, shape, 2)
    if l == 0:
        return (yi + 32 * (xi // 64)) * 64 + (xi % 64)
    if l == 1:
        return _OFF[1] + (yi + 8 * (xi // 32)) * 32 + (xi % 32)
    if l == 2:
        return _OFF[2] + (yi + 2 * (xi // 16)) * 16 + (xi % 16)
    return _OFF[3] + xi


def _body(*refs):
    t_cls = refs[0:5]
    s_cls = refs[5:10]
    t_bbox = refs[10:15]
    t_angle = refs[15:20]
    t_ctr = refs[20:25]
    s_bbox = refs[25:30]
    s_angle = refs[30:35]
    s_ctr = refs[35:40]
    out_cls, out_bbox, out_ctr = refs[40:43]
    v = refs[43:47]
    pnr = refs[47:51]
    pbb = refs[51:55]
    pct = refs[55:59]
    sdf = refs[59]

    i = pl.program_id(0)

    @pl.when(i == 0)
    def _init():
        sdf[1] = jnp.float32(0.0)

    @pl.when(i < _B)
    def _stream():
        b = i
        neg_acc = jnp.float32(0.0)
        rows_v = []
        rows_pnr = []
        rows_bb = []
        rows_ct = []
        for l in range(5):
            w_l = _SZ[l][1]
            g = min(128 // w_l, _C)
            cg = _C // g

            def widen(a):
                return jnp.concatenate(
                    [a[:, j * cg:(j + 1) * cg] for j in range(g)], axis=3)

            def fold(row, op):
                width = row.shape[2]
                while width > w_l:
                    width //= 2
                    row = op(row[:, :, :width], row[:, :, width:])
                return row

            tc = widen(t_cls[l][...])
            x = widen(s_cls[l][...])
            sp, p = _softplus_p(x)
            t = jax.nn.sigmoid(tc)
            d = t - p
            pos = (sp - t * x) * (d * d)
            neg = sp * (p * p)
            neg_acc = neg_acc + jnp.sum(neg)
            rows_v.append(jax.nn.sigmoid(
                fold(jnp.max(tc, axis=1), jnp.maximum)))
            rows_pnr.append(fold(jnp.sum(pos - neg, axis=1), jnp.add))

            sl1 = jnp.sum(_smooth_l1(s_bbox[l][...], t_bbox[l][...]), axis=1)
            sl1 = sl1 + _smooth_l1(s_angle[l][...][:, 0],
                                   t_angle[l][...][:, 0])
            w = jax.nn.sigmoid(t_ctr[l][...][:, 0])
            rows_bb.append(sl1 * w)

            xs = s_ctr[l][...][:, 0]
            es = jnp.exp(-jnp.abs(xs))
            sps = jnp.maximum(xs, 0.0) + jnp.log(1.0 + es)
            rows_ct.append(sps - w * xs)
        sdf[1] += neg_acc

        def flat(val):
            h, w = val.shape[1], val.shape[2]
            return jnp.concatenate(
                [val[:, j:j + 1, :] for j in range(h)], axis=2)

        for dst, rows, sent in ((v, rows_v, -1.0), (pnr, rows_pnr, 0.0),
                                (pbb, rows_bb, 0.0), (pct, rows_ct, 0.0)):
            for l in range(3):
                dst[l][pl.ds(b, 1)] = _pack_rows(rows[l], sent)
            tail = jnp.concatenate(
                [flat(rows[3]), flat(rows[4]),
                 jnp.full((1, 1, 48), sent, jnp.float32)], axis=2)
            dst[3][pl.ds(b, 1)] = tail

    @pl.when(i == _B)
    def _finish():
        keys = [lax.bitcast_convert_type(v[g][...], jnp.int32)
                for g in range(4)]

        def cnt_gt(x):
            c = jnp.int32(0)
            for k in keys:
                c = c + jnp.sum((k > x).astype(jnp.int32))
            return c

        def bstep(_, lohi):
            lo, hi = lohi
            mid = lo + (hi - lo) // 2
            take_hi = cnt_gt(mid) < _K
            return (jnp.where(take_hi, lo, mid), jnp.where(take_hi, mid, hi))

        _, t_key = lax.fori_loop(0, 31, bstep,
                                 (jnp.int32(-1), jnp.int32(0x3F800000)))
        r = _K - cnt_gt(t_key)

        idxs = [lax.broadcasted_iota(jnp.int32, keys[g].shape, 0) * _NPB
                + _packed_idx(keys[g].shape, g) for g in range(4)]

        def cnt_eq_le(x):
            c = jnp.int32(0)
            for k, ix in zip(keys, idxs):
                c = c + jnp.sum(((k == t_key) & (ix <= x)).astype(jnp.int32))
            return c

        def istep(_, lohi):
            lo, hi = lohi
            mid = lo + (hi - lo) // 2
            enough = cnt_eq_le(mid) >= r
            return (jnp.where(enough, lo, mid), jnp.where(enough, mid, hi))

        _, i_star = lax.fori_loop(0, 17, istep,
                                  (jnp.int32(-1), jnp.int32(_N - 1)))

        t_val = lax.bitcast_convert_type(t_key, jnp.float32)
        fg = t_val * r.astype(jnp.float32)
        acc_cls = jnp.float32(0.0)
        acc_bbox = jnp.float32(0.0)
        acc_ctr = jnp.float32(0.0)
        for g in range(4):
            gt = keys[g] > t_key
            mask = gt | ((keys[g] == t_key) & (idxs[g] <= i_star))
            fg = fg + jnp.sum(jnp.where(gt, v[g][...], 0.0))
            acc_cls = acc_cls + jnp.sum(jnp.where(mask, pnr[g][...], 0.0))
            acc_bbox = acc_bbox + jnp.sum(jnp.where(mask, pbb[g][...], 0.0))
            acc_ctr = acc_ctr + jnp.sum(jnp.where(mask, pct[g][...], 0.0))

        out_cls[0, 0] = (sdf[1] + acc_cls) / fg
        out_bbox[0, 0] = acc_bbox / jnp.float32(_K * 5)
        out_ctr[0, 0] = acc_ctr / jnp.float32(_K)


@jax.jit
def kernel(
    t_cls_0, t_cls_1, t_cls_2, t_cls_3, t_cls_4,
    t_bbox_0, t_bbox_1, t_bbox_2, t_bbox_3, t_bbox_4,
    t_angle_0, t_angle_1, t_angle_2, t_angle_3, t_angle_4,
    t_ctr_0, t_ctr_1, t_ctr_2, t_ctr_3, t_ctr_4,
    s_cls_0, s_cls_1, s_cls_2, s_cls_3, s_cls_4,
    s_bbox_0, s_bbox_1, s_bbox_2, s_bbox_3, s_bbox_4,
    s_angle_0, s_angle_1, s_angle_2, s_angle_3, s_angle_4,
    s_ctr_0, s_ctr_1, s_ctr_2, s_ctr_3, s_ctr_4,
):
    def bm4(i):
        return (jnp.clip(i, 0, _B - 1), 0, 0, 0)

    blk_cls = [pl.BlockSpec((1, _C, h, w), bm4) for h, w in _SZ]
    blk_bb = [pl.BlockSpec((1, 4, h, w), bm4) for h, w in _SZ]
    blk_1 = [pl.BlockSpec((1, 1, h, w), bm4) for h, w in _SZ]

    scr3 = [pltpu.VMEM((_B, 32, 128), jnp.float32),
            pltpu.VMEM((_B, 8, 128), jnp.float32),
            pltpu.VMEM((_B, 2, 128), jnp.float32),
            pltpu.VMEM((_B, 1, 128), jnp.float32)]

    loss_cls, loss_bbox, loss_ctr = pl.pallas_call(
        _body,
        grid=(_B + 1,),
        in_specs=(blk_cls + blk_cls + blk_bb + blk_1 + blk_1
                  + blk_bb + blk_1 + blk_1),
        out_specs=[pl.BlockSpec(memory_space=pltpu.SMEM)] * 3,
        out_shape=[jax.ShapeDtypeStruct((1, 1), jnp.float32)] * 3,
        scratch_shapes=(scr3 + scr3 + scr3 + scr3
                        + [pltpu.SMEM((4,), jnp.float32)]),
    )(t_cls_0, t_cls_1, t_cls_2, t_cls_3, t_cls_4,
      s_cls_0, s_cls_1, s_cls_2, s_cls_3, s_cls_4,
      t_bbox_0, t_bbox_1, t_bbox_2, t_bbox_3, t_bbox_4,
      t_angle_0, t_angle_1, t_angle_2, t_angle_3, t_angle_4,
      t_ctr_0, t_ctr_1, t_ctr_2, t_ctr_3, t_ctr_4,
      s_bbox_0, s_bbox_1, s_bbox_2, s_bbox_3, s_bbox_4,
      s_angle_0, s_angle_1, s_angle_2, s_angle_3, s_angle_4,
      s_ctr_0, s_ctr_1, s_ctr_2, s_ctr_3, s_ctr_4)
    return (loss_cls.reshape(()), loss_bbox.reshape(()), loss_ctr.reshape(()))


# speedup vs baseline: 1.0511x; 1.0511x over previous
import jax
import jax.numpy as jnp
from jax import lax
from jax.experimental import pallas as pl
from jax.experimental.pallas import tpu as pltpu

_B = 16
_C = 16
_SZ = ((64, 64), (32, 32), (16, 16), (8, 8), (4, 4))
_NPB = 5456
_N = _B * _NPB
_K = int(_N * 0.01)
_OFF = (0, 4096, 5120, 5376, 5440)


def _smooth_l1(x, y):
    d = jnp.abs(x - y)
    return jnp.where(d < 1.0, 0.5 * d * d, d - 0.5)


def _softplus_p(x):
    e = jnp.exp(-jnp.abs(x))
    a = 1.0 + e
    p = jnp.where(x >= 0.0, 1.0, e) / a
    sp = jnp.maximum(x, 0.0) + jnp.log(a)
    return sp, p


def _pack_rows(val, sent):
    del sent
    h = val.shape[1]
    w = val.shape[2]
    group = 128 // w
    pieces = [val[:, j * (h // group):(j + 1) * (h // group), :]
              for j in range(group)]
    return jnp.concatenate(pieces, axis=2)


def _packed_idx(shape, l):
    yi = lax.broadcasted_iota(jnp.int32, shape, 1)
    xi = lax.broadcasted_iota(jnp.int32, shape, 2)
    if l == 0:
        return (yi + 32 * (xi // 64)) * 64 + (xi % 64)
    if l == 1:
        return _OFF[1] + (yi + 8 * (xi // 32)) * 32 + (xi % 32)
    if l == 2:
        return _OFF[2] + (yi + 2 * (xi // 16)) * 16 + (xi % 16)
    return _OFF[3] + xi


def _body(*refs):
    t_cls = refs[0:5]
    s_cls = refs[5:10]
    t_bbox = refs[10:15]
    t_angle = refs[15:20]
    t_ctr = refs[20:25]
    s_bbox = refs[25:30]
    s_angle = refs[30:35]
    s_ctr = refs[35:40]
    out_cls, out_bbox, out_ctr = refs[40:43]
    v = refs[43:47]
    pnr = refs[47:51]
    pbb = refs[51:55]
    pct = refs[55:59]
    sdf = refs[59]

    i = pl.program_id(0)

    @pl.when(i == 0)
    def _init():
        sdf[1] = jnp.float32(0.0)

    @pl.when(i < _B)
    def _stream():
        b = i
        neg_acc = jnp.float32(0.0)
        rows_v = []
        rows_pnr = []
        rows_bb = []
        rows_ct = []
        for l in range(5):
            tc = t_cls[l][...]
            x = s_cls[l][...]
            sp, p = _softplus_p(x)
            t = jax.nn.sigmoid(tc)
            d = t - p
            pos = (sp - t * x) * (d * d)
            neg = sp * (p * p)
            neg_acc = neg_acc + jnp.sum(neg)
            rows_v.append(jax.nn.sigmoid(jnp.max(tc, axis=1)))
            rows_pnr.append(jnp.sum(pos - neg, axis=1))

            sl1 = jnp.sum(_smooth_l1(s_bbox[l][...], t_bbox[l][...]), axis=1)
            sl1 = sl1 + _smooth_l1(s_angle[l][...][:, 0],
                                   t_angle[l][...][:, 0])
            w = jax.nn.sigmoid(t_ctr[l][...][:, 0])
            rows_bb.append(sl1 * w)

            xs = s_ctr[l][...][:, 0]
            es = jnp.exp(-jnp.abs(xs))
            sps = jnp.maximum(xs, 0.0) + jnp.log(1.0 + es)
            rows_ct.append(sps - w * xs)
        sdf[1] += neg_acc

        def flat(val):
            h, w = val.shape[1], val.shape[2]
            return jnp.concatenate(
                [val[:, j:j + 1, :] for j in range(h)], axis=2)

        for dst, rows, sent in ((v, rows_v, -1.0), (pnr, rows_pnr, 0.0),
                                (pbb, rows_bb, 0.0), (pct, rows_ct, 0.0)):
            for l in range(3):
                dst[l][pl.ds(b, 1)] = _pack_rows(rows[l], sent)
            tail = jnp.concatenate(
                [flat(rows[3]), flat(rows[4]),
                 jnp.full((1, 1, 48), sent, jnp.float32)], axis=2)
            dst[3][pl.ds(b, 1)] = tail

    @pl.when(i == _B)
    def _finish():
        keys = [lax.bitcast_convert_type(v[g][...], jnp.int32)
                for g in range(4)]

        def cnt_gt(x):
            c = jnp.int32(0)
            for k in keys:
                c = c + jnp.sum((k > x).astype(jnp.int32))
            return c

        def bstep(_, lohi):
            lo, hi = lohi
            mid = lo + (hi - lo) // 2
            take_hi = cnt_gt(mid) < _K
            return (jnp.where(take_hi, lo, mid), jnp.where(take_hi, mid, hi))

        _, t_key = lax.fori_loop(0, 31, bstep,
                                 (jnp.int32(-1), jnp.int32(0x3F800000)))
        r = _K - cnt_gt(t_key)

        idxs = [lax.broadcasted_iota(jnp.int32, keys[g].shape, 0) * _NPB
                + _packed_idx(keys[g].shape, g) for g in range(4)]

        def cnt_eq_le(x):
            c = jnp.int32(0)
            for k, ix in zip(keys, idxs):
                c = c + jnp.sum(((k == t_key) & (ix <= x)).astype(jnp.int32))
            return c

        def istep(_, lohi):
            lo, hi = lohi
            mid = lo + (hi - lo) // 2
            enough = cnt_eq_le(mid) >= r
            return (jnp.where(enough, lo, mid), jnp.where(enough, mid, hi))

        _, i_star = lax.fori_loop(0, 17, istep,
                                  (jnp.int32(-1), jnp.int32(_N - 1)))

        t_val = lax.bitcast_convert_type(t_key, jnp.float32)
        fg = t_val * r.astype(jnp.float32)
        acc_cls = jnp.float32(0.0)
        acc_bbox = jnp.float32(0.0)
        acc_ctr = jnp.float32(0.0)
        for g in range(4):
            gt = keys[g] > t_key
            mask = gt | ((keys[g] == t_key) & (idxs[g] <= i_star))
            fg = fg + jnp.sum(jnp.where(gt, v[g][...], 0.0))
            acc_cls = acc_cls + jnp.sum(jnp.where(mask, pnr[g][...], 0.0))
            acc_bbox = acc_bbox + jnp.sum(jnp.where(mask, pbb[g][...], 0.0))
            acc_ctr = acc_ctr + jnp.sum(jnp.where(mask, pct[g][...], 0.0))

        out_cls[0, 0] = (sdf[1] + acc_cls) / fg
        out_bbox[0, 0] = acc_bbox / jnp.float32(_K * 5)
        out_ctr[0, 0] = acc_ctr / jnp.float32(_K)


@jax.jit
def kernel(
    t_cls_0, t_cls_1, t_cls_2, t_cls_3, t_cls_4,
    t_bbox_0, t_bbox_1, t_bbox_2, t_bbox_3, t_bbox_4,
    t_angle_0, t_angle_1, t_angle_2, t_angle_3, t_angle_4,
    t_ctr_0, t_ctr_1, t_ctr_2, t_ctr_3, t_ctr_4,
    s_cls_0, s_cls_1, s_cls_2, s_cls_3, s_cls_4,
    s_bbox_0, s_bbox_1, s_bbox_2, s_bbox_3, s_bbox_4,
    s_angle_0, s_angle_1, s_angle_2, s_angle_3, s_angle_4,
    s_ctr_0, s_ctr_1, s_ctr_2, s_ctr_3, s_ctr_4,
):
    def bm4(i):
        return (jnp.clip(i, 0, _B - 1), 0, 0, 0)

    blk_cls = [pl.BlockSpec((1, _C, h, w), bm4) for h, w in _SZ]
    blk_bb = [pl.BlockSpec((1, 4, h, w), bm4) for h, w in _SZ]
    blk_1 = [pl.BlockSpec((1, 1, h, w), bm4) for h, w in _SZ]

    scr3 = [pltpu.VMEM((_B, 32, 128), jnp.float32),
            pltpu.VMEM((_B, 8, 128), jnp.float32),
            pltpu.VMEM((_B, 2, 128), jnp.float32),
            pltpu.VMEM((_B, 1, 128), jnp.float32)]

    loss_cls, loss_bbox, loss_ctr = pl.pallas_call(
        _body,
        grid=(_B + 1,),
        in_specs=(blk_cls + blk_cls + blk_bb + blk_1 + blk_1
                  + blk_bb + blk_1 + blk_1),
        out_specs=[pl.BlockSpec(memory_space=pltpu.SMEM)] * 3,
        out_shape=[jax.ShapeDtypeStruct((1, 1), jnp.float32)] * 3,
        scratch_shapes=(scr3 + scr3 + scr3 + scr3
                        + [pltpu.SMEM((4,), jnp.float32)]),
    )(t_cls_0, t_cls_1, t_cls_2, t_cls_3, t_cls_4,
      s_cls_0, s_cls_1, s_cls_2, s_cls_3, s_cls_4,
      t_bbox_0, t_bbox_1, t_bbox_2, t_bbox_3, t_bbox_4,
      t_angle_0, t_angle_1, t_angle_2, t_angle_3, t_angle_4,
      t_ctr_0, t_ctr_1, t_ctr_2, t_ctr_3, t_ctr_4,
      s_bbox_0, s_bbox_1, s_bbox_2, s_bbox_3, s_bbox_4,
      s_angle_0, s_angle_1, s_angle_2, s_angle_3, s_angle_4,
      s_ctr_0, s_ctr_1, s_ctr_2, s_ctr_3, s_ctr_4)
    return (loss_cls.reshape(()), loss_bbox.reshape(()), loss_ctr.reshape(()))
